# baseline (speedup 1.0000x reference)
import functools
import math

import jax
import jax.numpy as jnp
from jax import lax
from jax.experimental import pallas as pl
from jax.experimental.pallas import tpu as pltpu
from jax.experimental.pallas import tpu_sc as plsc

N = 1048576
L_CNT = math.floor(N * 0.8)
H_CNT = N - L_CNT
BBITS = 16
SHIFT = 32 - BBITS
NB = 1 << BBITS
ROWS, COLS = NB // 512, 512
HP = 262144
DUMP = 212992
FROWS = 512

NC, NS = 2, 16
CHUNK = N // (NC * NS)
CROWS = CHUNK // 128
SLICE = NB // NS
ZB = 2048
MININT = -2147483648

@functools.cache
def _sc_mesh():
    return plsc.VectorSubcoreMesh(
        core_axis_name="c", subcore_axis_name="s",
        num_cores=NC, num_subcores=NS)


def _keys_row(xv, r):
    outs = []
    for j in range(8):
        u = plsc.bitcast(xv[pl.ds(r * 128 + j * 16, 16)], jnp.int32)
        k = u ^ (lax.shift_right_arithmetic(u, 31) | MININT)
        outs.append(lax.shift_right_logical(k, SHIFT))
    return outs


def _hist_body(x_hbm, out_hbm, xv, bidx, ones, zbuf, hist, sem):
    c = lax.axis_index("c")
    s = lax.axis_index("s")
    wid = c * NS + s

    pltpu.sync_copy(x_hbm.at[pl.ds(wid * CHUNK, CHUNK)], xv)

    z16 = jnp.zeros((16,), jnp.int32)
    for i in range(ZB // 16):
        zbuf[pl.ds(i * 16, 16)] = z16
    o16 = jnp.ones((16,), jnp.int32)
    for i in range(128 // 16):
        ones[pl.ds(i * 16, 16)] = o16

    def zloop(i, _):
        pltpu.sync_copy(zbuf, hist.at[pl.ds(s * SLICE + i * ZB, ZB)])
        return 0
    lax.fori_loop(0, SLICE // ZB, zloop, 0)

    plsc.subcore_barrier()

    NG = CROWS // 32

    def _compute_group(g):
        @plsc.parallel_loop(0, 32, unroll=4)
        def _(r):
            row = g * 32 + r
            for j, b in enumerate(_keys_row(xv, row)):
                bidx[row, pl.ds(j * 16, 16)] = b

    _compute_group(0)

    def gloop(g, _):
        descs = [pltpu.async_copy(
            ones, hist.at[bidx.at[(g - 1) * 32 + jj]], sem, add=True)
            for jj in range(32)]
        _compute_group(g)
        for d in descs:
            d.wait()
        return 0
    lax.fori_loop(1, NG, gloop, 0)

    last = [pltpu.async_copy(
        ones, hist.at[bidx.at[(NG - 1) * 32 + jj]], sem, add=True)
        for jj in range(32)]
    for d in last:
        d.wait()

    plsc.subcore_barrier()

    pltpu.sync_copy(hist.at[pl.ds(s * SLICE, SLICE)],
                    out_hbm.at[c, pl.ds(s * SLICE, SLICE)])


@functools.cache
def _hist_call():
    return pl.kernel(
        _hist_body,
        out_type=jax.ShapeDtypeStruct((NC, NB), jnp.int32),
        mesh=_sc_mesh(),
        compiler_params=pltpu.CompilerParams(needs_layout_passes=False),
        scratch_types=[
            pltpu.VMEM((CHUNK,), jnp.float32),
            pltpu.VMEM((CROWS, 128), jnp.int32),
            pltpu.VMEM((128,), jnp.int32),
            pltpu.VMEM((ZB,), jnp.int32),
            pltpu.VMEM_SHARED((NB,), jnp.int32),
            pltpu.SemaphoreType.DMA,
        ],
    )


def _shift_lanes(x, s, fill):
    pad = jnp.full((x.shape[0], s), fill, x.dtype)
    return jnp.concatenate([pad, x[:, : x.shape[1] - s]], axis=1)


def _shift_rows(x, s, fill):
    pad = jnp.full((s, x.shape[1]), fill, x.dtype)
    return jnp.concatenate([pad, x[: x.shape[0] - s, :]], axis=0)


NINF_BITS = -8388608
TBINS = NB // NS


def _scatter_body(hist_hbm, scat_hbm, h0_v, h1_v, cl_v, pos_v, val_v,
                  minf, tbuf, scat_sp, tot_sp, sem):
    c = lax.axis_index("c")
    s = lax.axis_index("s")
    iota16 = lax.iota(jnp.int32, 16)

    @pl.when(c == 0)
    def _():
        m16 = jnp.full((16,), NINF_BITS, jnp.int32)
        for i in range(ZB // 16):
            minf[pl.ds(i * 16, 16)] = m16

        def iloop(i, _):
            pltpu.sync_copy(
                minf, scat_sp.at[pl.ds(s * (HP // NS) + i * ZB, ZB)])
            return 0
        lax.fori_loop(0, HP // NS // ZB, iloop, 0)

        pltpu.sync_copy(hist_hbm.at[0, pl.ds(s * TBINS, TBINS)], h0_v)
        pltpu.sync_copy(hist_hbm.at[1, pl.ds(s * TBINS, TBINS)], h1_v)

        def sloop(i, carry):
            h = h0_v[pl.ds(i * 16, 16)] + h1_v[pl.ds(i * 16, 16)]
            h0_v[pl.ds(i * 16, 16)] = h
            cl_v[pl.ds(i * 16, 16)] = plsc.cumsum(h) + carry
            return carry + jnp.sum(h)
        total = lax.fori_loop(0, TBINS // 16, sloop, 0)

        minf[pl.ds(0, 16)] = jnp.where(iota16 == s, total, 0)
        pltpu.sync_copy(minf.at[pl.ds(0, 16)], tot_sp.at[pl.ds(s * 16, 16)])

        plsc.subcore_barrier()

        pltpu.sync_copy(tot_sp, tbuf)
        tvec = jnp.zeros((16,), jnp.int32)
        for t in range(NS):
            tvec = tvec + tbuf[pl.ds(t * 16, 16)]
        prefix = jnp.sum(jnp.where(iota16 < s, tvec, 0))

        @plsc.parallel_loop(0, TBINS // 16, unroll=4)
        def _(i):
            h = h0_v[pl.ds(i * 16, 16)]
            C = cl_v[pl.ds(i * 16, 16)] + prefix
            b = s * TBINS + i * 16 + iota16
            mask = (h > 0) & (C > L_CNT)
            posv = jnp.where(mask, N - C, DUMP + (b & 8191))
            kc = lax.shift_left(b, SHIFT) | (1 << (SHIFT - 1))
            bits = jnp.where(kc < 0, kc ^ MININT, ~kc)
            r = lax.shift_right_logical(i, 3)
            col = (i & 7) * 16
            pos_v[r, pl.ds(col, 16)] = posv
            val_v[r, pl.ds(col, 16)] = bits ^ MININT

        descs = []
        for row in range(TBINS // 128):
            descs.append(pltpu.async_copy(
                val_v.at[row], scat_sp.at[pos_v.at[row]], sem))
        for d in descs:
            d.wait()

        plsc.subcore_barrier()

        rdescs = []
        for r in range(FROWS // NS):
            row = s * (FROWS // NS) + r
            rdescs.append(pltpu.async_copy(
                scat_sp.at[pl.ds((s * (FROWS // NS) + r) * FROWS, FROWS)],
                scat_hbm.at[row], sem))
        for d in rdescs:
            d.wait()


@functools.cache
def _scatter_call():
    return pl.kernel(
        _scatter_body,
        out_type=jax.ShapeDtypeStruct((FROWS, FROWS), jnp.int32),
        mesh=_sc_mesh(),
        compiler_params=pltpu.CompilerParams(needs_layout_passes=False),
        scratch_types=[
            pltpu.VMEM((TBINS,), jnp.int32),
            pltpu.VMEM((TBINS,), jnp.int32),
            pltpu.VMEM((TBINS,), jnp.int32),
            pltpu.VMEM((TBINS // 128, 128), jnp.int32),
            pltpu.VMEM((TBINS // 128, 128), jnp.int32),
            pltpu.VMEM((ZB,), jnp.int32),
            pltpu.VMEM((NS * 16,), jnp.int32),
            pltpu.VMEM_SHARED((HP,), jnp.int32),
            pltpu.VMEM_SHARED((NS * 16,), jnp.int32),
            pltpu.SemaphoreType.DMA,
        ],
    )


def _fill_body(s_ref, o_ref):
    xm = lax.bitcast_convert_type(s_ref[...], jnp.float32)
    ninf = float("-inf")
    for sh in (1, 2, 4, 8, 16, 32, 64, 128, 256):
        xm = jnp.maximum(xm, _shift_lanes(xm, sh, ninf))
    acc = xm[:, FROWS - 1 : FROWS]
    for sh in (1, 2, 4, 8, 16, 32, 64, 128, 256):
        acc = jnp.maximum(acc, _shift_rows(acc, sh, ninf))
    ex = _shift_rows(acc, 1, ninf)
    filled = jnp.maximum(xm, ex)
    t = jnp.log1p(filled * filled * 0.5)
    o_ref[...] = t.reshape(-1)[:H_CNT]


_fill_call = pl.pallas_call(
    _fill_body,
    out_shape=jax.ShapeDtypeStruct((H_CNT,), jnp.float32),
)


def kernel(ub_log_conf):
    hist = _hist_call()(ub_log_conf)
    scat = _scatter_call()(hist)
    return _fill_call(scat)


# speedup vs baseline: 1.0004x; 1.0004x over previous
import functools
import math

import jax
import jax.numpy as jnp
from jax import lax
from jax.experimental import pallas as pl
from jax.experimental.pallas import tpu as pltpu
from jax.experimental.pallas import tpu_sc as plsc

N = 1048576
L_CNT = math.floor(N * 0.8)
H_CNT = N - L_CNT
BBITS = 16
SHIFT = 32 - BBITS
NB = 1 << BBITS
HP = 262144
DUMP = 212992
FROWS = 512

NC, NS = 2, 16
CHUNK = N // (NC * NS)
CROWS = CHUNK // 128
SLICE = NB // NS
ZB = 2048
MININT = -2147483648

@functools.cache
def _sc_mesh():
    return plsc.VectorSubcoreMesh(
        core_axis_name="c", subcore_axis_name="s",
        num_cores=NC, num_subcores=NS)


def _keys_row(xv, r):
    outs = []
    for j in range(8):
        u = plsc.bitcast(xv[pl.ds(r * 128 + j * 16, 16)], jnp.int32)
        k = u ^ (lax.shift_right_arithmetic(u, 31) | MININT)
        outs.append(lax.shift_right_logical(k, SHIFT))
    return outs


def _hist_body(x_hbm, out_hbm, xv, bidx, ones, zbuf, hist, sem):
    c = lax.axis_index("c")
    s = lax.axis_index("s")
    wid = c * NS + s

    pltpu.sync_copy(x_hbm.at[pl.ds(wid * CHUNK, CHUNK)], xv)

    z16 = jnp.zeros((16,), jnp.int32)
    for i in range(ZB // 16):
        zbuf[pl.ds(i * 16, 16)] = z16
    o16 = jnp.ones((16,), jnp.int32)
    for i in range(128 // 16):
        ones[pl.ds(i * 16, 16)] = o16

    def zloop(i, _):
        pltpu.sync_copy(zbuf, hist.at[pl.ds(s * SLICE + i * ZB, ZB)])
        return 0
    lax.fori_loop(0, SLICE // ZB, zloop, 0)

    plsc.subcore_barrier()

    NG = CROWS // 32

    def _compute_group(g):
        @plsc.parallel_loop(0, 32, unroll=4)
        def _(r):
            row = g * 32 + r
            for j, b in enumerate(_keys_row(xv, row)):
                bidx[row, pl.ds(j * 16, 16)] = b

    _compute_group(0)

    def gloop(g, _):
        descs = [pltpu.async_copy(
            ones, hist.at[bidx.at[(g - 1) * 32 + jj]], sem, add=True)
            for jj in range(32)]
        _compute_group(g)
        for d in descs:
            d.wait()
        return 0
    lax.fori_loop(1, NG, gloop, 0)

    last = [pltpu.async_copy(
        ones, hist.at[bidx.at[(NG - 1) * 32 + jj]], sem, add=True)
        for jj in range(32)]
    for d in last:
        d.wait()

    plsc.subcore_barrier()

    pltpu.sync_copy(hist.at[pl.ds(s * SLICE, SLICE)],
                    out_hbm.at[c, pl.ds(s * SLICE, SLICE)])


@functools.cache
def _hist_call():
    return pl.kernel(
        _hist_body,
        out_type=jax.ShapeDtypeStruct((NC, NB), jnp.int32),
        mesh=_sc_mesh(),
        compiler_params=pltpu.CompilerParams(needs_layout_passes=False),
        scratch_types=[
            pltpu.VMEM((CHUNK,), jnp.float32),
            pltpu.VMEM((CROWS, 128), jnp.int32),
            pltpu.VMEM((128,), jnp.int32),
            pltpu.VMEM((ZB,), jnp.int32),
            pltpu.VMEM_SHARED((NB,), jnp.int32),
            pltpu.SemaphoreType.DMA,
        ],
    )


def _shift_lanes(x, s, fill):
    pad = jnp.full((x.shape[0], s), fill, x.dtype)
    return jnp.concatenate([pad, x[:, : x.shape[1] - s]], axis=1)


def _shift_rows(x, s, fill):
    pad = jnp.full((s, x.shape[1]), fill, x.dtype)
    return jnp.concatenate([pad, x[: x.shape[0] - s, :]], axis=0)


NINF_BITS = -8388608
TBINS = NB // NS


def _scatter_body(hist_hbm, scat_hbm, h0_v, h1_v, cl_v, pos_v, val_v,
                  minf, tbuf, scat_sp, tot_sp, sem):
    c = lax.axis_index("c")
    s = lax.axis_index("s")
    iota16 = lax.iota(jnp.int32, 16)

    @pl.when(c == 0)
    def _():
        m16 = jnp.full((16,), NINF_BITS, jnp.int32)
        for i in range(ZB // 16):
            minf[pl.ds(i * 16, 16)] = m16

        def iloop(i, _):
            pltpu.sync_copy(
                minf, scat_sp.at[pl.ds(s * (HP // NS) + i * ZB, ZB)])
            return 0
        lax.fori_loop(0, HP // NS // ZB, iloop, 0)

        pltpu.sync_copy(hist_hbm.at[0, pl.ds(s * TBINS, TBINS)], h0_v)
        pltpu.sync_copy(hist_hbm.at[1, pl.ds(s * TBINS, TBINS)], h1_v)

        def sloop(i, carry):
            h = h0_v[pl.ds(i * 16, 16)] + h1_v[pl.ds(i * 16, 16)]
            h0_v[pl.ds(i * 16, 16)] = h
            cl_v[pl.ds(i * 16, 16)] = plsc.cumsum(h) + carry
            return carry + jnp.sum(h)
        total = lax.fori_loop(0, TBINS // 16, sloop, 0)

        minf[pl.ds(0, 16)] = jnp.where(iota16 == s, total, 0)
        pltpu.sync_copy(minf.at[pl.ds(0, 16)], tot_sp.at[pl.ds(s * 16, 16)])

        plsc.subcore_barrier()

        pltpu.sync_copy(tot_sp, tbuf)
        tvec = jnp.zeros((16,), jnp.int32)
        for t in range(NS):
            tvec = tvec + tbuf[pl.ds(t * 16, 16)]
        prefix = jnp.sum(jnp.where(iota16 < s, tvec, 0))

        @plsc.parallel_loop(0, TBINS // 16, unroll=4)
        def _(i):
            h = h0_v[pl.ds(i * 16, 16)]
            C = cl_v[pl.ds(i * 16, 16)] + prefix
            b = s * TBINS + i * 16 + iota16
            mask = (h > 0) & (C > L_CNT)
            posv = jnp.where(mask, N - C, DUMP + (b & 8191))
            kc = lax.shift_left(b, SHIFT) | (1 << (SHIFT - 1))
            bits = jnp.where(kc < 0, kc ^ MININT, ~kc)
            r = lax.shift_right_logical(i, 3)
            col = (i & 7) * 16
            pos_v[r, pl.ds(col, 16)] = posv
            val_v[r, pl.ds(col, 16)] = bits ^ MININT

        descs = []
        for row in range(TBINS // 128):
            descs.append(pltpu.async_copy(
                val_v.at[row], scat_sp.at[pos_v.at[row]], sem))
        for d in descs:
            d.wait()

        plsc.subcore_barrier()

        rdescs = []
        for r in range(FROWS // NS):
            row = s * (FROWS // NS) + r
            rdescs.append(pltpu.async_copy(
                scat_sp.at[pl.ds((s * (FROWS // NS) + r) * FROWS, FROWS)],
                scat_hbm.at[row], sem))
        for d in rdescs:
            d.wait()


@functools.cache
def _scatter_call():
    return pl.kernel(
        _scatter_body,
        out_type=jax.ShapeDtypeStruct((FROWS, FROWS), jnp.int32),
        mesh=_sc_mesh(),
        compiler_params=pltpu.CompilerParams(needs_layout_passes=False),
        scratch_types=[
            pltpu.VMEM((TBINS,), jnp.int32),
            pltpu.VMEM((TBINS,), jnp.int32),
            pltpu.VMEM((TBINS,), jnp.int32),
            pltpu.VMEM((TBINS // 128, 128), jnp.int32),
            pltpu.VMEM((TBINS // 128, 128), jnp.int32),
            pltpu.VMEM((ZB,), jnp.int32),
            pltpu.VMEM((NS * 16,), jnp.int32),
            pltpu.VMEM_SHARED((HP,), jnp.int32),
            pltpu.VMEM_SHARED((NS * 16,), jnp.int32),
            pltpu.SemaphoreType.DMA,
        ],
    )


def _fill_body(s_ref, o_ref):
    xm = lax.bitcast_convert_type(s_ref[...], jnp.float32)
    ninf = float("-inf")
    for sh in (1, 2, 4, 8, 16, 32, 64, 128, 256):
        xm = jnp.maximum(xm, _shift_lanes(xm, sh, ninf))
    acc = xm[:, FROWS - 1 : FROWS]
    for sh in (1, 2, 4, 8, 16, 32, 64, 128, 256):
        acc = jnp.maximum(acc, _shift_rows(acc, sh, ninf))
    ex = _shift_rows(acc, 1, ninf)
    filled = jnp.maximum(xm, ex)
    t = jnp.log1p(filled * filled * 0.5)
    o_ref[...] = t.reshape(-1)[:H_CNT]


_fill_call = pl.pallas_call(
    _fill_body,
    out_shape=jax.ShapeDtypeStruct((H_CNT,), jnp.float32),
)


def kernel(ub_log_conf):
    hist = _hist_call()(ub_log_conf)
    scat = _scatter_call()(hist)
    return _fill_call(scat)


# speedup vs baseline: 1.0410x; 1.0405x over previous
import functools
import math

import jax
import jax.numpy as jnp
from jax import lax
from jax.experimental import pallas as pl
from jax.experimental.pallas import tpu as pltpu
from jax.experimental.pallas import tpu_sc as plsc

N = 1048576
L_CNT = math.floor(N * 0.8)
H_CNT = N - L_CNT
BBITS = 16
SHIFT = 32 - BBITS
NB = 1 << BBITS
HP = 262144
DUMP = 212992
FROWS = 512

NC, NS = 2, 16
CHUNK = N // (NC * NS)
CROWS = CHUNK // 128
SLICE = NB // NS
ZB = 2048
MININT = -2147483648

@functools.cache
def _sc_mesh():
    return plsc.VectorSubcoreMesh(
        core_axis_name="c", subcore_axis_name="s",
        num_cores=NC, num_subcores=NS)


def _keys_row(xv, r):
    outs = []
    for j in range(8):
        u = plsc.bitcast(xv[pl.ds(r * 128 + j * 16, 16)], jnp.int32)
        k = u ^ (lax.shift_right_arithmetic(u, 31) | MININT)
        outs.append(lax.shift_right_logical(k, SHIFT))
    return outs


def _hist_body(x_hbm, out_hbm, xv, bidx, ones, zbuf, hist, sem):
    c = lax.axis_index("c")
    s = lax.axis_index("s")
    wid = c * NS + s

    xdesc = pltpu.async_copy(x_hbm.at[pl.ds(wid * CHUNK, CHUNK)], xv, sem)

    z16 = jnp.zeros((16,), jnp.int32)
    for i in range(ZB // 16):
        zbuf[pl.ds(i * 16, 16)] = z16
    o16 = jnp.ones((16,), jnp.int32)
    for i in range(128 // 16):
        ones[pl.ds(i * 16, 16)] = o16

    def zloop(i, _):
        pltpu.sync_copy(zbuf, hist.at[pl.ds(s * SLICE + i * ZB, ZB)])
        return 0
    lax.fori_loop(0, SLICE // ZB, zloop, 0)

    xdesc.wait()
    plsc.subcore_barrier()

    NG = CROWS // 32

    def _compute_group(g):
        @plsc.parallel_loop(0, 32, unroll=4)
        def _(r):
            row = g * 32 + r
            for j, b in enumerate(_keys_row(xv, row)):
                bidx[row, pl.ds(j * 16, 16)] = b

    _compute_group(0)

    def gloop(g, _):
        descs = [pltpu.async_copy(
            ones, hist.at[bidx.at[(g - 1) * 32 + jj]], sem, add=True)
            for jj in range(32)]
        _compute_group(g)
        for d in descs:
            d.wait()
        return 0
    lax.fori_loop(1, NG, gloop, 0)

    last = [pltpu.async_copy(
        ones, hist.at[bidx.at[(NG - 1) * 32 + jj]], sem, add=True)
        for jj in range(32)]
    for d in last:
        d.wait()

    plsc.subcore_barrier()

    pltpu.sync_copy(hist.at[pl.ds(s * SLICE, SLICE)],
                    out_hbm.at[c, pl.ds(s * SLICE, SLICE)])


@functools.cache
def _hist_call():
    return pl.kernel(
        _hist_body,
        out_type=jax.ShapeDtypeStruct((NC, NB), jnp.int32),
        mesh=_sc_mesh(),
        compiler_params=pltpu.CompilerParams(needs_layout_passes=False),
        scratch_types=[
            pltpu.VMEM((CHUNK,), jnp.float32),
            pltpu.VMEM((CROWS, 128), jnp.int32),
            pltpu.VMEM((128,), jnp.int32),
            pltpu.VMEM((ZB,), jnp.int32),
            pltpu.VMEM_SHARED((NB,), jnp.int32),
            pltpu.SemaphoreType.DMA,
        ],
    )


def _shift_lanes(x, s, fill):
    pad = jnp.full((x.shape[0], s), fill, x.dtype)
    return jnp.concatenate([pad, x[:, : x.shape[1] - s]], axis=1)


def _shift_rows(x, s, fill):
    pad = jnp.full((s, x.shape[1]), fill, x.dtype)
    return jnp.concatenate([pad, x[: x.shape[0] - s, :]], axis=0)


NINF_BITS = -8388608
TBINS = NB // NS


def _scatter_body(hist_hbm, scat_hbm, h0_v, h1_v, cl_v, pos_v, val_v,
                  minf, tbuf, scat_sp, tot_sp, sem):
    c = lax.axis_index("c")
    s = lax.axis_index("s")
    iota16 = lax.iota(jnp.int32, 16)

    @pl.when(c == 0)
    def _():
        hdesc0 = pltpu.async_copy(
            hist_hbm.at[0, pl.ds(s * TBINS, TBINS)], h0_v, sem)
        hdesc1 = pltpu.async_copy(
            hist_hbm.at[1, pl.ds(s * TBINS, TBINS)], h1_v, sem)

        m16 = jnp.full((16,), NINF_BITS, jnp.int32)
        for i in range(ZB // 16):
            minf[pl.ds(i * 16, 16)] = m16

        def iloop(i, _):
            pltpu.sync_copy(
                minf, scat_sp.at[pl.ds(s * (HP // NS) + i * ZB, ZB)])
            return 0
        lax.fori_loop(0, HP // NS // ZB, iloop, 0)

        hdesc0.wait()
        hdesc1.wait()

        def sloop(i, carry):
            h = h0_v[pl.ds(i * 16, 16)] + h1_v[pl.ds(i * 16, 16)]
            h0_v[pl.ds(i * 16, 16)] = h
            cl_v[pl.ds(i * 16, 16)] = plsc.cumsum(h) + carry
            return carry + jnp.sum(h)
        total = lax.fori_loop(0, TBINS // 16, sloop, 0)

        minf[pl.ds(0, 16)] = jnp.where(iota16 == s, total, 0)
        pltpu.sync_copy(minf.at[pl.ds(0, 16)], tot_sp.at[pl.ds(s * 16, 16)])

        plsc.subcore_barrier()

        pltpu.sync_copy(tot_sp, tbuf)
        tvec = jnp.zeros((16,), jnp.int32)
        for t in range(NS):
            tvec = tvec + tbuf[pl.ds(t * 16, 16)]
        prefix = jnp.sum(jnp.where(iota16 < s, tvec, 0))

        @plsc.parallel_loop(0, TBINS // 16, unroll=4)
        def _(i):
            h = h0_v[pl.ds(i * 16, 16)]
            C = cl_v[pl.ds(i * 16, 16)] + prefix
            b = s * TBINS + i * 16 + iota16
            mask = (h > 0) & (C > L_CNT)
            posv = jnp.where(mask, N - C, DUMP + (b & 8191))
            kc = lax.shift_left(b, SHIFT) | (1 << (SHIFT - 1))
            bits = jnp.where(kc < 0, kc ^ MININT, ~kc)
            r = lax.shift_right_logical(i, 3)
            col = (i & 7) * 16
            pos_v[r, pl.ds(col, 16)] = posv
            val_v[r, pl.ds(col, 16)] = bits ^ MININT

        descs = []
        for row in range(TBINS // 128):
            descs.append(pltpu.async_copy(
                val_v.at[row], scat_sp.at[pos_v.at[row]], sem))
        for d in descs:
            d.wait()

        plsc.subcore_barrier()

        rdescs = []
        for r in range(FROWS // NS):
            row = s * (FROWS // NS) + r
            rdescs.append(pltpu.async_copy(
                scat_sp.at[pl.ds((s * (FROWS // NS) + r) * FROWS, FROWS)],
                scat_hbm.at[row], sem))
        for d in rdescs:
            d.wait()


@functools.cache
def _scatter_call():
    return pl.kernel(
        _scatter_body,
        out_type=jax.ShapeDtypeStruct((FROWS, FROWS), jnp.int32),
        mesh=_sc_mesh(),
        compiler_params=pltpu.CompilerParams(needs_layout_passes=False),
        scratch_types=[
            pltpu.VMEM((TBINS,), jnp.int32),
            pltpu.VMEM((TBINS,), jnp.int32),
            pltpu.VMEM((TBINS,), jnp.int32),
            pltpu.VMEM((TBINS // 128, 128), jnp.int32),
            pltpu.VMEM((TBINS // 128, 128), jnp.int32),
            pltpu.VMEM((ZB,), jnp.int32),
            pltpu.VMEM((NS * 16,), jnp.int32),
            pltpu.VMEM_SHARED((HP,), jnp.int32),
            pltpu.VMEM_SHARED((NS * 16,), jnp.int32),
            pltpu.SemaphoreType.DMA,
        ],
    )


def _fill_body(s_ref, o_ref):
    xm = lax.bitcast_convert_type(s_ref[...], jnp.float32)
    ninf = float("-inf")
    for sh in (1, 2, 4, 8, 16, 32, 64, 128, 256):
        xm = jnp.maximum(xm, _shift_lanes(xm, sh, ninf))
    acc = xm[:, FROWS - 1 : FROWS]
    for sh in (1, 2, 4, 8, 16, 32, 64, 128, 256):
        acc = jnp.maximum(acc, _shift_rows(acc, sh, ninf))
    ex = _shift_rows(acc, 1, ninf)
    filled = jnp.maximum(xm, ex)
    t = jnp.log1p(filled * filled * 0.5)
    o_ref[...] = t.reshape(-1)[:H_CNT]


_fill_call = pl.pallas_call(
    _fill_body,
    out_shape=jax.ShapeDtypeStruct((H_CNT,), jnp.float32),
)


def kernel(ub_log_conf):
    hist = _hist_call()(ub_log_conf)
    scat = _scatter_call()(hist)
    return _fill_call(scat)
